# baseline (speedup 1.0000x reference)
import functools

import jax
import jax.numpy as jnp
from jax import lax
from jax.experimental import pallas as pl
from jax.experimental.pallas import tpu as pltpu
from jax.experimental.pallas import tpu_sc as plsc

_THRESHOLD = 0.1
_NC = 2
_NS = 16
_L = 16
_CHUNK = 512


def _sc_body(x_hbm, t_hbm, out_hbm, xbuf, tbuf, cntbuf, *, num_classes,
             batch, graph):
    wid = lax.axis_index("s") * _NC + lax.axis_index("c")
    nw = _NC * _NS
    batches_per_w = batch // nw
    chunks_per_b = graph // _CHUNK
    n_chunks = batches_per_w * chunks_per_b

    def chunk_body(c, cnt):
        b = wid * batches_per_w + c // chunks_per_b
        g0 = (c % chunks_per_b) * _CHUNK
        pltpu.sync_copy(x_hbm.at[b, :, pl.ds(g0, _CHUNK)], xbuf)
        pltpu.sync_copy(t_hbm.at[b, pl.ds(g0, _CHUNK)], tbuf)

        def lane_body(j, cnt):
            m1 = jnp.full((_L,), -jnp.inf, jnp.float32)
            m2 = jnp.full((_L,), -jnp.inf, jnp.float32)
            i1 = jnp.zeros((_L,), jnp.int32)
            i2 = jnp.zeros((_L,), jnp.int32)
            for k in range(num_classes):
                v = xbuf[k, pl.ds(j * _L, _L)]
                gt1 = v > m1
                gt2 = v > m2
                m2 = jnp.where(gt1, m1, jnp.where(gt2, v, m2))
                i2 = jnp.where(gt1, i1, jnp.where(gt2, k, i2))
                m1 = jnp.where(gt1, v, m1)
                i1 = jnp.where(gt1, k, i1)
            tv = tbuf[pl.ds(j * _L, _L)]
            one = jnp.ones((_L,), jnp.int32)
            zero = jnp.zeros((_L,), jnp.int32)
            c1 = jnp.where(i1 == tv, one, zero)
            sec = jnp.logical_and(m1 - m2 < _THRESHOLD, i2 == tv)
            c2 = jnp.where(sec, one, zero)
            return cnt + c1 + c2

        return lax.fori_loop(0, _CHUNK // _L, lane_body, cnt)

    cnt = lax.fori_loop(0, n_chunks, chunk_body, jnp.zeros((_L,), jnp.int32))
    cntbuf[...] = cnt
    pltpu.sync_copy(cntbuf, out_hbm.at[wid])


def kernel(input, target):
    batch, num_classes, graph = input.shape
    nw = _NC * _NS

    body = functools.partial(
        _sc_body, num_classes=num_classes, batch=batch, graph=graph)
    partials = pl.kernel(
        body,
        out_type=jax.ShapeDtypeStruct((nw, _L), jnp.int32),
        scratch_types=[
            pltpu.VMEM((num_classes, _CHUNK), jnp.float32),
            pltpu.VMEM((_CHUNK,), jnp.int32),
            pltpu.VMEM((_L,), jnp.int32),
        ],
        mesh=plsc.VectorSubcoreMesh(core_axis_name="c", subcore_axis_name="s"),
    )(input, target)

    edge_acc = jnp.sum(partials).astype(jnp.float32) / float(target.size)
    return 1.0 - edge_acc


# speedup vs baseline: 1.2389x; 1.2389x over previous
import functools

import jax
import jax.numpy as jnp
from jax import lax
from jax.experimental import pallas as pl
from jax.experimental.pallas import tpu as pltpu
from jax.experimental.pallas import tpu_sc as plsc

_THRESHOLD = 0.1
_NC = 2
_NS = 16
_L = 16
_CHUNK = 1024


def _sc_body(x_hbm, t_hbm, out_hbm, xbuf0, xbuf1, tbuf0, tbuf1, cntbuf,
             sem0, sem1, *, num_classes, batch, graph):
    wid = lax.axis_index("s") * _NC + lax.axis_index("c")
    nw = _NC * _NS
    batches_per_w = batch // nw
    chunks_per_b = graph // _CHUNK
    n_chunks = batches_per_w * chunks_per_b

    def start_chunk(c, xb, tb, sem):
        b = wid * batches_per_w + c // chunks_per_b
        g0 = (c % chunks_per_b) * _CHUNK
        pltpu.make_async_copy(x_hbm.at[b, :, pl.ds(g0, _CHUNK)], xb, sem).start()
        pltpu.make_async_copy(t_hbm.at[b, pl.ds(g0, _CHUNK)], tb, sem).start()

    def wait_chunk(xb, tb, sem):
        pltpu.make_async_copy(x_hbm.at[0, :, pl.ds(0, _CHUNK)], xb, sem).wait()
        pltpu.make_async_copy(t_hbm.at[0, pl.ds(0, _CHUNK)], tb, sem).wait()

    def compute(xb, tb, cnt):
        def lane_body(j, cnt):
            m1 = jnp.full((_L,), -jnp.inf, jnp.float32)
            m2 = jnp.full((_L,), -jnp.inf, jnp.float32)
            i1 = jnp.zeros((_L,), jnp.int32)
            i2 = jnp.zeros((_L,), jnp.int32)
            for k in range(num_classes):
                v = xb[k, pl.ds(j * _L, _L)]
                gt1 = v > m1
                gt2 = v > m2
                m2 = jnp.where(gt1, m1, jnp.where(gt2, v, m2))
                i2 = jnp.where(gt1, i1, jnp.where(gt2, k, i2))
                m1 = jnp.where(gt1, v, m1)
                i1 = jnp.where(gt1, k, i1)
            tv = tb[pl.ds(j * _L, _L)]
            one = jnp.ones((_L,), jnp.int32)
            zero = jnp.zeros((_L,), jnp.int32)
            c1 = jnp.where(i1 == tv, one, zero)
            sec = jnp.logical_and(m1 - m2 < _THRESHOLD, i2 == tv)
            c2 = jnp.where(sec, one, zero)
            return cnt + c1 + c2

        return lax.fori_loop(0, _CHUNK // _L, lane_body, cnt)

    start_chunk(0, xbuf0, tbuf0, sem0)
    start_chunk(1, xbuf1, tbuf1, sem1)

    def pair_body(p, cnt):
        c = 2 * p
        wait_chunk(xbuf0, tbuf0, sem0)
        cnt = compute(xbuf0, tbuf0, cnt)

        @pl.when(c + 2 < n_chunks)
        def _():
            start_chunk(c + 2, xbuf0, tbuf0, sem0)

        wait_chunk(xbuf1, tbuf1, sem1)
        cnt = compute(xbuf1, tbuf1, cnt)

        @pl.when(c + 3 < n_chunks)
        def _():
            start_chunk(c + 3, xbuf1, tbuf1, sem1)

        return cnt

    cnt = lax.fori_loop(0, n_chunks // 2, pair_body,
                        jnp.zeros((_L,), jnp.int32))
    cntbuf[...] = cnt
    pltpu.sync_copy(cntbuf, out_hbm.at[wid])


def kernel(input, target):
    batch, num_classes, graph = input.shape
    nw = _NC * _NS

    body = functools.partial(
        _sc_body, num_classes=num_classes, batch=batch, graph=graph)
    partials = pl.kernel(
        body,
        out_type=jax.ShapeDtypeStruct((nw, _L), jnp.int32),
        scratch_types=[
            pltpu.VMEM((num_classes, _CHUNK), jnp.float32),
            pltpu.VMEM((num_classes, _CHUNK), jnp.float32),
            pltpu.VMEM((_CHUNK,), jnp.int32),
            pltpu.VMEM((_CHUNK,), jnp.int32),
            pltpu.VMEM((_L,), jnp.int32),
            pltpu.SemaphoreType.DMA,
            pltpu.SemaphoreType.DMA,
        ],
        mesh=plsc.VectorSubcoreMesh(core_axis_name="c", subcore_axis_name="s"),
    )(input, target)

    edge_acc = jnp.sum(partials).astype(jnp.float32) / float(target.size)
    return 1.0 - edge_acc


# speedup vs baseline: 1.4596x; 1.1781x over previous
import functools

import jax
import jax.numpy as jnp
from jax import lax
from jax.experimental import pallas as pl
from jax.experimental.pallas import tpu as pltpu
from jax.experimental.pallas import tpu_sc as plsc

_THRESHOLD = 0.1
_NC = 2
_NS = 16
_L = 16
_CHUNK = 1024


def _sc_body(x_hbm, t_hbm, out_hbm, xbuf0, xbuf1, tbuf0, tbuf1, cntbuf,
             sem0, sem1, *, num_classes, batch, graph):
    wid = lax.axis_index("s") * _NC + lax.axis_index("c")
    nw = _NC * _NS
    batches_per_w = batch // nw
    chunks_per_b = graph // _CHUNK
    n_chunks = batches_per_w * chunks_per_b

    def start_chunk(c, xb, tb, sem):
        b = wid * batches_per_w + c // chunks_per_b
        g0 = (c % chunks_per_b) * _CHUNK
        pltpu.make_async_copy(x_hbm.at[b, :, pl.ds(g0, _CHUNK)], xb, sem).start()
        pltpu.make_async_copy(t_hbm.at[b, pl.ds(g0, _CHUNK)], tb, sem).start()

    def wait_chunk(xb, tb, sem):
        pltpu.make_async_copy(x_hbm.at[0, :, pl.ds(0, _CHUNK)], xb, sem).wait()
        pltpu.make_async_copy(t_hbm.at[0, pl.ds(0, _CHUNK)], tb, sem).wait()

    def compute(xb, tb, cnt):
        def lane_body(j, cnt):
            m1 = jnp.full((_L,), -jnp.inf, jnp.float32)
            m2 = jnp.full((_L,), -jnp.inf, jnp.float32)
            i1 = jnp.zeros((_L,), jnp.int32)
            i2 = jnp.zeros((_L,), jnp.int32)
            for k in range(num_classes):
                v = xb[k, pl.ds(j * _L, _L)]
                gt1 = v > m1
                gt2 = v > m2
                i2 = jnp.where(gt1, i1, jnp.where(gt2, k, i2))
                i1 = jnp.where(gt1, k, i1)
                m2 = jnp.maximum(m2, jnp.minimum(v, m1))
                m1 = jnp.maximum(v, m1)
            tv = tb[pl.ds(j * _L, _L)]
            one = jnp.ones((_L,), jnp.int32)
            zero = jnp.zeros((_L,), jnp.int32)
            c1 = jnp.where(i1 == tv, one, zero)
            sec = jnp.logical_and(m1 - m2 < _THRESHOLD, i2 == tv)
            c2 = jnp.where(sec, one, zero)
            return cnt + c1 + c2

        return lax.fori_loop(0, _CHUNK // _L, lane_body, cnt)

    start_chunk(0, xbuf0, tbuf0, sem0)
    start_chunk(1, xbuf1, tbuf1, sem1)

    def pair_body(p, cnt):
        c = 2 * p
        wait_chunk(xbuf0, tbuf0, sem0)
        cnt = compute(xbuf0, tbuf0, cnt)

        @pl.when(c + 2 < n_chunks)
        def _():
            start_chunk(c + 2, xbuf0, tbuf0, sem0)

        wait_chunk(xbuf1, tbuf1, sem1)
        cnt = compute(xbuf1, tbuf1, cnt)

        @pl.when(c + 3 < n_chunks)
        def _():
            start_chunk(c + 3, xbuf1, tbuf1, sem1)

        return cnt

    cnt = lax.fori_loop(0, n_chunks // 2, pair_body,
                        jnp.zeros((_L,), jnp.int32))
    cntbuf[...] = cnt
    pltpu.sync_copy(cntbuf, out_hbm.at[wid])


def kernel(input, target):
    batch, num_classes, graph = input.shape
    nw = _NC * _NS

    body = functools.partial(
        _sc_body, num_classes=num_classes, batch=batch, graph=graph)
    partials = pl.kernel(
        body,
        out_type=jax.ShapeDtypeStruct((nw, _L), jnp.int32),
        scratch_types=[
            pltpu.VMEM((num_classes, _CHUNK), jnp.float32),
            pltpu.VMEM((num_classes, _CHUNK), jnp.float32),
            pltpu.VMEM((_CHUNK,), jnp.int32),
            pltpu.VMEM((_CHUNK,), jnp.int32),
            pltpu.VMEM((_L,), jnp.int32),
            pltpu.SemaphoreType.DMA,
            pltpu.SemaphoreType.DMA,
        ],
        mesh=plsc.VectorSubcoreMesh(core_axis_name="c", subcore_axis_name="s"),
    )(input, target)

    edge_acc = jnp.sum(partials).astype(jnp.float32) / float(target.size)
    return 1.0 - edge_acc
